# Initial kernel scaffold; baseline (speedup 1.0000x reference)
import jax
import jax.numpy as jnp
from jax.experimental import pallas as pl


def kernel(hidden_states, weight):
    raise NotImplementedError("write your pallas kernel here")


# speedup vs baseline: 1.0916x; 1.0916x over previous
import jax
import jax.numpy as jnp
from jax.experimental import pallas as pl
from jax.experimental.pallas import tpu as pltpu

_HIDDEN = 4096
_EXPERTS = 64
_TOPK = 8
_BLOCK_T = 512


def _router_block_kernel(hs_ref, w_ref, logits_ref, scores_ref, idx_ref):
    hs = hs_ref[...]
    w = w_ref[...]
    logits = jax.lax.dot_general(
        hs, w, (((1,), (1,)), ((), ())),
        preferred_element_type=jnp.float32)

    m = jnp.max(logits, axis=-1, keepdims=True)
    e = jnp.exp(logits - m)
    p = e / jnp.sum(e, axis=-1, keepdims=True)
    logits_ref[...] = p

    iota = jax.lax.broadcasted_iota(jnp.int32, p.shape, 1)
    x = p
    vals = []
    idxs = []
    for _ in range(_TOPK):
        mk = jnp.max(x, axis=-1, keepdims=True)
        is_max = x == mk
        ik = jnp.min(jnp.where(is_max, iota, _EXPERTS), axis=-1, keepdims=True)
        vals.append(mk)
        idxs.append(ik)
        x = jnp.where(iota == ik, -1.0, x)
    topv = jnp.concatenate(vals, axis=-1)
    topi = jnp.concatenate(idxs, axis=-1)
    scores_ref[...] = topv / jnp.sum(topv, axis=-1, keepdims=True)
    idx_ref[...] = topi


def kernel(hidden_states, weight):
    hs = hidden_states.reshape(-1, _HIDDEN)
    n_tokens = hs.shape[0]
    grid = (n_tokens // _BLOCK_T,)

    logits, scores, indices = pl.pallas_call(
        _router_block_kernel,
        grid=grid,
        in_specs=[
            pl.BlockSpec((_BLOCK_T, _HIDDEN), lambda i: (i, 0)),
            pl.BlockSpec((_EXPERTS, _HIDDEN), lambda i: (0, 0)),
        ],
        out_specs=[
            pl.BlockSpec((_BLOCK_T, _EXPERTS), lambda i: (i, 0)),
            pl.BlockSpec((_BLOCK_T, _TOPK), lambda i: (i, 0)),
            pl.BlockSpec((_BLOCK_T, _TOPK), lambda i: (i, 0)),
        ],
        out_shape=[
            jax.ShapeDtypeStruct((n_tokens, _EXPERTS), jnp.float32),
            jax.ShapeDtypeStruct((n_tokens, _TOPK), jnp.float32),
            jax.ShapeDtypeStruct((n_tokens, _TOPK), jnp.int32),
        ],
    )(hs, weight)
    return (logits, scores, indices)


# speedup vs baseline: 1.2977x; 1.1888x over previous
import jax
import jax.numpy as jnp
from jax.experimental import pallas as pl
from jax.experimental.pallas import tpu as pltpu

_HIDDEN = 4096
_EXPERTS = 64
_TOPK = 8
_BLOCK_T = 512


def _router_block_kernel(hs_ref, w_ref, logits_ref, scores_ref, idx_ref):
    hs = hs_ref[...]
    w = w_ref[...]
    logits = jax.lax.dot_general(
        hs, w, (((1,), (1,)), ((), ())),
        preferred_element_type=jnp.float32)

    m = jnp.max(logits, axis=-1, keepdims=True)
    e = jnp.exp(logits - m)
    p = e / jnp.sum(e, axis=-1, keepdims=True)
    logits_ref[...] = p

    iota = jax.lax.broadcasted_iota(jnp.int32, p.shape, 1)
    pbits = jax.lax.bitcast_convert_type(p, jnp.int32)
    key = jax.lax.bitcast_convert_type(
        (pbits & ~0x3F) | (0x3F - iota), jnp.float32)
    vals = []
    idxs = []
    for _ in range(_TOPK):
        mk = jnp.max(key, axis=-1, keepdims=True)
        key = jnp.where(key == mk, -1.0, key)
        mbits = jax.lax.bitcast_convert_type(mk, jnp.int32)
        idxs.append(0x3F - (mbits & 0x3F))
        vals.append(jax.lax.bitcast_convert_type(
            (mbits & ~0x3F) | 0x20, jnp.float32))
    topv = jnp.concatenate(vals, axis=-1)
    topi = jnp.concatenate(idxs, axis=-1)
    scores_ref[...] = topv / jnp.sum(topv, axis=-1, keepdims=True)
    idx_ref[...] = topi


def kernel(hidden_states, weight):
    hs = hidden_states.reshape(-1, _HIDDEN)
    n_tokens = hs.shape[0]
    grid = (n_tokens // _BLOCK_T,)

    logits, scores, indices = pl.pallas_call(
        _router_block_kernel,
        grid=grid,
        in_specs=[
            pl.BlockSpec((_BLOCK_T, _HIDDEN), lambda i: (i, 0)),
            pl.BlockSpec((_EXPERTS, _HIDDEN), lambda i: (0, 0)),
        ],
        out_specs=[
            pl.BlockSpec((_BLOCK_T, _EXPERTS), lambda i: (i, 0)),
            pl.BlockSpec((_BLOCK_T, _TOPK), lambda i: (i, 0)),
            pl.BlockSpec((_BLOCK_T, _TOPK), lambda i: (i, 0)),
        ],
        out_shape=[
            jax.ShapeDtypeStruct((n_tokens, _EXPERTS), jnp.float32),
            jax.ShapeDtypeStruct((n_tokens, _TOPK), jnp.float32),
            jax.ShapeDtypeStruct((n_tokens, _TOPK), jnp.int32),
        ],
    )(hs, weight)
    return (logits, scores, indices)


# speedup vs baseline: 1.4145x; 1.0900x over previous
import jax
import jax.numpy as jnp
from jax.experimental import pallas as pl
from jax.experimental.pallas import tpu as pltpu

_HIDDEN = 4096
_EXPERTS = 64
_TOPK = 8
_BLOCK_T = 1024


def _router_block_kernel(hs_ref, w_ref, logits_ref, scores_ref, idx_ref):
    hs = hs_ref[...]
    w = w_ref[...]
    logits = jax.lax.dot_general(
        hs, w, (((1,), (1,)), ((), ())),
        preferred_element_type=jnp.float32)

    m = jnp.max(logits, axis=-1, keepdims=True)
    e = jnp.exp(logits - m)
    p = e / jnp.sum(e, axis=-1, keepdims=True)
    logits_ref[...] = p

    iota = jax.lax.broadcasted_iota(jnp.int32, p.shape, 1)
    pbits = jax.lax.bitcast_convert_type(p, jnp.int32)
    key = jax.lax.bitcast_convert_type(
        (pbits & ~0x3F) | (0x3F - iota), jnp.float32)
    vals = []
    idxs = []
    for _ in range(_TOPK):
        mk = jnp.max(key, axis=-1, keepdims=True)
        key = jnp.where(key == mk, -1.0, key)
        mbits = jax.lax.bitcast_convert_type(mk, jnp.int32)
        idxs.append(0x3F - (mbits & 0x3F))
        vals.append(jax.lax.bitcast_convert_type(
            (mbits & ~0x3F) | 0x20, jnp.float32))
    topv = jnp.concatenate(vals, axis=-1)
    topi = jnp.concatenate(idxs, axis=-1)
    scores_ref[...] = topv / jnp.sum(topv, axis=-1, keepdims=True)
    idx_ref[...] = topi


def kernel(hidden_states, weight):
    hs = hidden_states.reshape(-1, _HIDDEN)
    n_tokens = hs.shape[0]
    grid = (n_tokens // _BLOCK_T,)

    logits, scores, indices = pl.pallas_call(
        _router_block_kernel,
        grid=grid,
        in_specs=[
            pl.BlockSpec((_BLOCK_T, _HIDDEN), lambda i: (i, 0)),
            pl.BlockSpec((_EXPERTS, _HIDDEN), lambda i: (0, 0)),
        ],
        out_specs=[
            pl.BlockSpec((_BLOCK_T, _EXPERTS), lambda i: (i, 0)),
            pl.BlockSpec((_BLOCK_T, _TOPK), lambda i: (i, 0)),
            pl.BlockSpec((_BLOCK_T, _TOPK), lambda i: (i, 0)),
        ],
        out_shape=[
            jax.ShapeDtypeStruct((n_tokens, _EXPERTS), jnp.float32),
            jax.ShapeDtypeStruct((n_tokens, _TOPK), jnp.float32),
            jax.ShapeDtypeStruct((n_tokens, _TOPK), jnp.int32),
        ],
    )(hs, weight)
    return (logits, scores, indices)
